# Initial kernel scaffold; baseline (speedup 1.0000x reference)
import jax
import jax.numpy as jnp
from jax.experimental import pallas as pl


def kernel(x, edge_index, W1_self, W1_neigh, b1, W2_self, W2_neigh, b2, Wc, bc):
    raise NotImplementedError("write your pallas kernel here")


# speedup vs baseline: 6.1218x; 6.1218x over previous
import functools

import jax
import jax.numpy as jnp
from jax import lax
from jax.experimental import pallas as pl
from jax.experimental.pallas import tpu as pltpu
from jax.experimental.pallas import tpu_sc as plsc

_CH = 128


def _make_sc_agg(n_pad, d_row, n_edges, with_deg):
    assert n_edges % _CH == 0
    n_chunks = n_edges // _CH
    mesh = plsc.VectorSubcoreMesh(core_axis_name="c", subcore_axis_name="s")
    nc, ns = mesh.num_cores, mesh.num_subcores
    nw = nc * ns
    rows_per_tile = n_pad // ns
    assert rows_per_tile % _CH == 0
    n_base, n_rem = n_chunks // nw, n_chunks % nw

    out_type = [jax.ShapeDtypeStruct((nc, n_pad, d_row), jnp.float32)]
    scratch = [
        pltpu.VMEM((_CH,), jnp.int32),
        pltpu.VMEM((_CH,), jnp.int32),
        pltpu.VMEM((_CH, d_row), jnp.float32),
        pltpu.VMEM_SHARED((n_pad, d_row), jnp.float32),
        pltpu.SemaphoreType.DMA,
    ]
    if with_deg:
        out_type.append(jax.ShapeDtypeStruct((nc, ns, n_pad), jnp.float32))
        scratch.append(pltpu.VMEM((n_pad,), jnp.float32))

    @functools.partial(
        pl.kernel, out_type=out_type, mesh=mesh, scratch_types=scratch,
        compiler_params=pltpu.CompilerParams(needs_layout_passes=False))
    def sc_agg(feat_hbm, src_hbm, dst_hbm, out_hbm, *rest):
        if with_deg:
            deg_hbm, src_buf, dst_buf, rows_buf, acc, sem, deg_v = rest
        else:
            src_buf, dst_buf, rows_buf, acc, sem = rest
        c = lax.axis_index("c")
        s = lax.axis_index("s")

        def zero_row(i, _):
            for k in range(d_row // 16):
                rows_buf[i, pl.ds(k * 16, 16)] = jnp.zeros((16,), jnp.float32)
            return 0

        lax.fori_loop(0, _CH, zero_row, 0)
        for m in range(rows_per_tile // _CH):
            pltpu.sync_copy(
                rows_buf, acc.at[pl.ds(s * rows_per_tile + m * _CH, _CH)])
        if with_deg:
            def zero_deg(i, _):
                deg_v[pl.ds(i * 16, 16)] = jnp.zeros((16,), jnp.float32)
                return 0
            lax.fori_loop(0, n_pad // 16, zero_deg, 0)
        plsc.subcore_barrier()

        wid = s * nc + c
        n_my = n_base + jnp.where(wid < n_rem, 1, 0)

        def body(j, _):
            base = (wid + nw * j) * _CH
            pltpu.sync_copy(src_hbm.at[pl.ds(base, _CH)], src_buf)
            pltpu.sync_copy(dst_hbm.at[pl.ds(base, _CH)], dst_buf)
            pltpu.async_copy(feat_hbm.at[src_buf], rows_buf, sem).wait()
            pltpu.sync_copy(rows_buf, acc.at[dst_buf], add=True)
            if with_deg:
                for k in range(_CH // 16):
                    v = dst_buf[pl.ds(k * 16, 16)]
                    cnt, last = plsc.scan_count(v)
                    plsc.addupdate_scatter(
                        deg_v, [v], cnt.astype(jnp.float32), mask=last)
            return 0

        lax.fori_loop(0, n_my, body, 0)
        plsc.subcore_barrier()

        for m in range(rows_per_tile // _CH):
            off = s * rows_per_tile + m * _CH
            pltpu.sync_copy(acc.at[pl.ds(off, _CH)],
                            out_hbm.at[c, pl.ds(off, _CH)])
        if with_deg:
            pltpu.sync_copy(deg_v, deg_hbm.at[c, s])

    return sc_agg


def _mean_h(x_ref, aa_ref, ab_ref, deg_ref, ws_ref, wn_ref, b_ref):
    deg = jnp.sum(deg_ref[...], axis=0)
    inv = (1.0 / jnp.maximum(deg, 1.0))[:, None]
    mean = (aa_ref[...] + ab_ref[...]) * inv
    h = (jnp.dot(x_ref[...], ws_ref[...], preferred_element_type=jnp.float32)
         + jnp.dot(mean, wn_ref[...], preferred_element_type=jnp.float32)
         + b_ref[...])
    return jnp.maximum(h, 0.0)


def _dense1_body(x_ref, aa_ref, ab_ref, deg_ref, ws_ref, wn_ref, b_ref,
                 o_ref):
    o_ref[...] = _mean_h(x_ref, aa_ref, ab_ref, deg_ref, ws_ref, wn_ref,
                         b_ref)


def _dense2_body(x_ref, aa_ref, ab_ref, deg_ref, ws_ref, wn_ref, b_ref,
                 wc_ref, bc_ref, o_ref):
    h = _mean_h(x_ref, aa_ref, ab_ref, deg_ref, ws_ref, wn_ref, b_ref)
    o_ref[...] = (jnp.dot(h, wc_ref[...], preferred_element_type=jnp.float32)
                  + bc_ref[...])


def _dense(body, n_pad, nw, d, h, extra_w, r=1024):
    grid = n_pad // r
    row_spec = pl.BlockSpec((r, d), lambda i: (i, 0))
    full = lambda shape: pl.BlockSpec(shape, lambda i: (0,) * len(shape))
    in_specs = [row_spec, row_spec, row_spec,
                pl.BlockSpec((nw, r), lambda i: (0, i)),
                full((d, h)), full((d, h)), full((1, h))]
    out_d = h
    for w in extra_w:
        in_specs += [full(w[0]), full(w[1])]
        out_d = w[0][1]
    return pl.pallas_call(
        body,
        grid=(grid,),
        in_specs=in_specs,
        out_specs=pl.BlockSpec((r, out_d), lambda i: (i, 0)),
        out_shape=jax.ShapeDtypeStruct((n_pad, out_d), jnp.float32),
    )


def kernel(x, edge_index, W1_self, W1_neigh, b1, W2_self, W2_neigh, b2,
           Wc, bc):
    n, d = x.shape
    h_dim = W1_self.shape[1]
    o_dim = W2_self.shape[1]
    c_dim = Wc.shape[1]
    e = edge_index.shape[1]

    n_pad = -(-n // 2048) * 2048
    e_pad = -(-e // _CH) * _CH

    src = edge_index[0].astype(jnp.int32)
    dst = edge_index[1].astype(jnp.int32)
    if e_pad != e:
        pad = e_pad - e
        src = jnp.concatenate([src, jnp.zeros((pad,), jnp.int32)])
        dst = jnp.concatenate(
            [dst, jnp.full((pad,), n_pad - 1, jnp.int32)])

    agg1, degp = _make_sc_agg(n_pad, d, e_pad, True)(x, src, dst)
    nw = degp.shape[0] * degp.shape[1]
    degp = degp.reshape(nw, n_pad)

    x_pad = jnp.zeros((n_pad, d), jnp.float32).at[:n].set(x)
    h1 = _dense(_dense1_body, n_pad, nw, d, h_dim, [])(
        x_pad, agg1[0], agg1[1], degp, W1_self, W1_neigh,
        b1.reshape(1, h_dim))

    (agg2,) = _make_sc_agg(n_pad, h_dim, e_pad, False)(h1, src, dst)

    out = _dense(_dense2_body, n_pad, nw, h_dim, o_dim,
                 [((o_dim, c_dim), (1, c_dim))])(
        h1, agg2[0], agg2[1], degp, W2_self, W2_neigh,
        b2.reshape(1, o_dim), Wc, bc.reshape(1, c_dim))
    return out[:n]


# speedup vs baseline: 11.1808x; 1.8264x over previous
import functools

import jax
import jax.numpy as jnp
from jax import lax
from jax.experimental import pallas as pl
from jax.experimental.pallas import tpu as pltpu
from jax.experimental.pallas import tpu_sc as plsc

_CH = 128


def _make_sc_agg(n_pad, d_row, n_edges, with_deg):
    assert n_edges % _CH == 0
    n_chunks = n_edges // _CH
    mesh = plsc.VectorSubcoreMesh(core_axis_name="c", subcore_axis_name="s")
    nc, ns = mesh.num_cores, mesh.num_subcores
    nw = nc * ns
    rows_per_tile = n_pad // ns
    assert rows_per_tile % _CH == 0
    n_base, n_rem = n_chunks // nw, n_chunks % nw
    nb = 2
    la = 1

    out_type = [jax.ShapeDtypeStruct((nc, n_pad, d_row), jnp.float32)]
    scratch = [
        *[pltpu.VMEM((_CH,), jnp.int32) for _ in range(nb)],
        *[pltpu.VMEM((_CH,), jnp.int32) for _ in range(nb)],
        *[pltpu.VMEM((_CH, d_row), jnp.float32) for _ in range(nb)],
        pltpu.VMEM_SHARED((n_pad, d_row), jnp.float32),
        *[pltpu.SemaphoreType.DMA for _ in range(4 * nb)],
    ]
    if with_deg:
        out_type.append(jax.ShapeDtypeStruct((nc, ns, n_pad), jnp.float32))
        scratch.append(pltpu.VMEM((n_pad,), jnp.float32))

    @functools.partial(
        pl.kernel, out_type=out_type, mesh=mesh, scratch_types=scratch,
        compiler_params=pltpu.CompilerParams(needs_layout_passes=False))
    def sc_agg(feat_hbm, src_hbm, dst_hbm, out_hbm, *rest):
        if with_deg:
            deg_hbm, *rest = rest
            deg_v = rest[-1]
            rest = rest[:-1]
        src_bufs = rest[:nb]
        dst_bufs = rest[nb:2 * nb]
        rows = rest[2 * nb:3 * nb]
        acc = rest[3 * nb]
        sems = rest[3 * nb + 1:]
        sem_j, sem_i, sem_g, sem_s = (sems[:nb], sems[nb:2 * nb],
                                      sems[2 * nb:3 * nb], sems[3 * nb:])
        c = lax.axis_index("c")
        s = lax.axis_index("s")

        def zero_row(i, _):
            for k in range(d_row // 16):
                rows[0][i, pl.ds(k * 16, 16)] = jnp.zeros((16,), jnp.float32)
            return 0

        lax.fori_loop(0, _CH, zero_row, 0)
        for m in range(rows_per_tile // _CH):
            pltpu.sync_copy(
                rows[0], acc.at[pl.ds(s * rows_per_tile + m * _CH, _CH)])
        if with_deg:
            def zero_deg(i, _):
                deg_v[pl.ds(i * 16, 16)] = jnp.zeros((16,), jnp.float32)
                return 0
            lax.fori_loop(0, n_pad // 16, zero_deg, 0)
        plsc.subcore_barrier()

        wid = s * nc + c
        n_my = n_base + jnp.where(wid < n_rem, 1, 0)
        start = wid * n_base + jnp.minimum(wid, n_rem)
        base_e = start * _CH

        def sidx_at(j, b):
            return pltpu.make_async_copy(
                src_hbm.at[pl.ds(base_e + j * _CH, _CH)], src_bufs[b],
                sem_j[b])

        def gather_at(j, b):
            return pltpu.make_async_copy(
                feat_hbm.at[src_bufs[b]], rows[b], sem_g[b])

        def idx_at(j, b):
            return pltpu.make_async_copy(
                dst_hbm.at[pl.ds(base_e + j * _CH, _CH)], dst_bufs[b],
                sem_i[b])

        def scat_at(j, b):
            return pltpu.make_async_copy(rows[b], acc.at[dst_bufs[b]],
                                         sem_s[b])

        def switch(b, fn):
            for bb in range(nb):
                @pl.when(b == bb)
                def _():
                    fn(bb)

        def body(t, _):
            @pl.when(t < n_my)
            def _issue():
                def go(bb):
                    @pl.when(t >= nb)
                    def _():
                        scat_at(t, bb).wait()
                    idx_at(t, bb).start()
                    sidx_at(t, bb).start()
                    sidx_at(t, bb).wait()
                    gather_at(t, bb).start()
                switch(t % nb, go)

            @pl.when(t >= la)
            def _consume():
                j = t - la

                def go(bb):
                    gather_at(j, bb).wait()
                    idx_at(j, bb).wait()
                    scat_at(j, bb).start(add=True)
                    if with_deg:
                        for k in range(_CH // 16):
                            v = dst_bufs[bb][pl.ds(k * 16, 16)]
                            cnt, last = plsc.scan_count(v)
                            plsc.addupdate_scatter(
                                deg_v, [v], cnt.astype(jnp.float32),
                                mask=last)
                switch(j % nb, go)
            return 0

        lax.fori_loop(0, n_my + la, body, 0)
        for bb in range(nb):
            @pl.when(bb < n_my)
            def _():
                scat_at(0, bb).wait()

        plsc.subcore_barrier()

        for m in range(rows_per_tile // _CH):
            off = s * rows_per_tile + m * _CH
            pltpu.sync_copy(acc.at[pl.ds(off, _CH)],
                            out_hbm.at[c, pl.ds(off, _CH)])
        if with_deg:
            pltpu.sync_copy(deg_v, deg_hbm.at[c, s])

    return sc_agg


def _mean_h(x_ref, aa_ref, ab_ref, deg_ref, ws_ref, wn_ref, b_ref):
    deg = jnp.sum(deg_ref[...], axis=0)
    inv = (1.0 / jnp.maximum(deg, 1.0))[:, None]
    mean = (aa_ref[...] + ab_ref[...]) * inv
    h = (jnp.dot(x_ref[...], ws_ref[...], preferred_element_type=jnp.float32)
         + jnp.dot(mean, wn_ref[...], preferred_element_type=jnp.float32)
         + b_ref[...])
    return jnp.maximum(h, 0.0)


def _dense1_body(x_ref, aa_ref, ab_ref, deg_ref, ws_ref, wn_ref, b_ref,
                 o_ref):
    o_ref[...] = _mean_h(x_ref, aa_ref, ab_ref, deg_ref, ws_ref, wn_ref,
                         b_ref)


def _dense2_body(x_ref, aa_ref, ab_ref, deg_ref, ws_ref, wn_ref, b_ref,
                 wc_ref, bc_ref, o_ref):
    h = _mean_h(x_ref, aa_ref, ab_ref, deg_ref, ws_ref, wn_ref, b_ref)
    o_ref[...] = (jnp.dot(h, wc_ref[...], preferred_element_type=jnp.float32)
                  + bc_ref[...])


def _dense(body, n_pad, nw, d, h, extra_w, r=1024):
    grid = n_pad // r
    row_spec = pl.BlockSpec((r, d), lambda i: (i, 0))
    full = lambda shape: pl.BlockSpec(shape, lambda i: (0,) * len(shape))
    in_specs = [row_spec, row_spec, row_spec,
                pl.BlockSpec((nw, r), lambda i: (0, i)),
                full((d, h)), full((d, h)), full((1, h))]
    out_d = h
    for w in extra_w:
        in_specs += [full(w[0]), full(w[1])]
        out_d = w[0][1]
    return pl.pallas_call(
        body,
        grid=(grid,),
        in_specs=in_specs,
        out_specs=pl.BlockSpec((r, out_d), lambda i: (i, 0)),
        out_shape=jax.ShapeDtypeStruct((n_pad, out_d), jnp.float32),
    )


def kernel(x, edge_index, W1_self, W1_neigh, b1, W2_self, W2_neigh, b2,
           Wc, bc):
    n, d = x.shape
    h_dim = W1_self.shape[1]
    o_dim = W2_self.shape[1]
    c_dim = Wc.shape[1]
    e = edge_index.shape[1]

    n_pad = -(-n // 2048) * 2048
    e_pad = -(-e // _CH) * _CH

    src = edge_index[0].astype(jnp.int32)
    dst = edge_index[1].astype(jnp.int32)
    pad = e_pad - e
    src = jnp.concatenate([src, jnp.zeros((pad + _CH,), jnp.int32)])
    dst = jnp.concatenate(
        [dst, jnp.full((pad + _CH,), n_pad - 1, jnp.int32)])

    agg1, degp = _make_sc_agg(n_pad, d, e_pad, True)(x, src, dst)
    nw = degp.shape[0] * degp.shape[1]
    degp = degp.reshape(nw, n_pad)

    x_pad = jnp.zeros((n_pad, d), jnp.float32).at[:n].set(x)
    h1 = _dense(_dense1_body, n_pad, nw, d, h_dim, [])(
        x_pad, agg1[0], agg1[1], degp, W1_self, W1_neigh,
        b1.reshape(1, h_dim))

    (agg2,) = _make_sc_agg(n_pad, h_dim, e_pad, False)(h1, src, dst)

    out = _dense(_dense2_body, n_pad, nw, h_dim, o_dim,
                 [((o_dim, c_dim), (1, c_dim))])(
        h1, agg2[0], agg2[1], degp, W2_self, W2_neigh,
        b2.reshape(1, o_dim), Wc, bc.reshape(1, c_dim))
    return out[:n]


# speedup vs baseline: 12.3376x; 1.1035x over previous
import functools

import jax
import jax.numpy as jnp
from jax import lax
from jax.experimental import pallas as pl
from jax.experimental.pallas import tpu as pltpu
from jax.experimental.pallas import tpu_sc as plsc

_CH = 128


def _make_sc_agg(n_pad, d_row, n_edges, with_deg):
    assert n_edges % _CH == 0
    n_chunks = n_edges // _CH
    mesh = plsc.VectorSubcoreMesh(core_axis_name="c", subcore_axis_name="s")
    nc, ns = mesh.num_cores, mesh.num_subcores
    nw = nc * ns
    rows_per_tile = n_pad // ns
    assert rows_per_tile % _CH == 0
    n_base, n_rem = n_chunks // nw, n_chunks % nw
    nb = 2
    la = 1
    pnb = 4
    pd = pnb - nb

    out_type = [jax.ShapeDtypeStruct((nc, n_pad, d_row), jnp.float32)]
    scratch = [
        *[pltpu.VMEM((_CH,), jnp.int32) for _ in range(pnb)],
        *[pltpu.VMEM((_CH,), jnp.int32) for _ in range(pnb)],
        *[pltpu.VMEM((_CH, d_row), jnp.float32) for _ in range(nb)],
        pltpu.VMEM_SHARED((n_pad, d_row), jnp.float32),
        *[pltpu.SemaphoreType.DMA for _ in range(2 * pnb + 2 * nb)],
    ]
    if with_deg:
        out_type.append(jax.ShapeDtypeStruct((nc, ns, n_pad), jnp.float32))
        scratch.append(pltpu.VMEM((n_pad,), jnp.float32))

    @functools.partial(
        pl.kernel, out_type=out_type, mesh=mesh, scratch_types=scratch,
        compiler_params=pltpu.CompilerParams(needs_layout_passes=False))
    def sc_agg(feat_hbm, src_hbm, dst_hbm, out_hbm, *rest):
        if with_deg:
            deg_hbm, *rest = rest
            deg_v = rest[-1]
            rest = rest[:-1]
        src_bufs = rest[:pnb]
        dst_bufs = rest[pnb:2 * pnb]
        rows = rest[2 * pnb:2 * pnb + nb]
        acc = rest[2 * pnb + nb]
        sems = rest[2 * pnb + nb + 1:]
        sem_j, sem_i, sem_g, sem_s = (sems[:pnb], sems[pnb:2 * pnb],
                                      sems[2 * pnb:2 * pnb + nb],
                                      sems[2 * pnb + nb:])
        c = lax.axis_index("c")
        s = lax.axis_index("s")

        def zero_row(i, _):
            for k in range(d_row // 16):
                rows[0][i, pl.ds(k * 16, 16)] = jnp.zeros((16,), jnp.float32)
            return 0

        lax.fori_loop(0, _CH, zero_row, 0)
        for m in range(rows_per_tile // _CH):
            pltpu.sync_copy(
                rows[0], acc.at[pl.ds(s * rows_per_tile + m * _CH, _CH)])
        if with_deg:
            def zero_deg(i, _):
                deg_v[pl.ds(i * 16, 16)] = jnp.zeros((16,), jnp.float32)
                return 0
            lax.fori_loop(0, n_pad // 16, zero_deg, 0)
        plsc.subcore_barrier()

        wid = s * nc + c
        n_my = n_base + jnp.where(wid < n_rem, 1, 0)
        start = wid * n_base + jnp.minimum(wid, n_rem)
        base_e = start * _CH

        def sidx_at(j, q):
            return pltpu.make_async_copy(
                src_hbm.at[pl.ds(base_e + j * _CH, _CH)], src_bufs[q],
                sem_j[q])

        def gather_at(q, b):
            return pltpu.make_async_copy(
                feat_hbm.at[src_bufs[q]], rows[b], sem_g[b])

        def idx_at(j, q):
            return pltpu.make_async_copy(
                dst_hbm.at[pl.ds(base_e + j * _CH, _CH)], dst_bufs[q],
                sem_i[q])

        def scat_at(q, b):
            return pltpu.make_async_copy(rows[b], acc.at[dst_bufs[q]],
                                         sem_s[b])

        def switch(q, fn):
            for qq in range(pnb):
                @pl.when(q == qq)
                def _():
                    fn(qq)

        for u in range(pd):
            @pl.when(u < n_my)
            def _():
                sidx_at(u, u).start()
                idx_at(u, u).start()

        def body(t, _):
            @pl.when(t < n_my)
            def _issue():
                @pl.when(t >= nb)
                def _():
                    def go_w(qq):
                        scat_at(qq, qq % nb).wait()
                    switch((t - nb) % pnb, go_w)

                u = t + pd

                @pl.when(u < n_my)
                def _prefetch():
                    def go_pf(qq):
                        sidx_at(u, qq).start()
                        idx_at(u, qq).start()
                    switch(u % pnb, go_pf)

                def go_g(qq):
                    sidx_at(t, qq).wait()
                    gather_at(qq, qq % nb).start()
                switch(t % pnb, go_g)

            @pl.when(t >= la)
            def _consume():
                j = t - la

                def go(qq):
                    idx_at(j, qq).wait()
                    gather_at(qq, qq % nb).wait()
                    scat_at(qq, qq % nb).start(add=True)
                    if with_deg:
                        for k in range(_CH // 16):
                            v = dst_bufs[qq][pl.ds(k * 16, 16)]
                            cnt, last = plsc.scan_count(v)
                            plsc.addupdate_scatter(
                                deg_v, [v], cnt.astype(jnp.float32),
                                mask=last)
                switch(j % pnb, go)
            return 0

        lax.fori_loop(0, n_my + la, body, 0)
        for d in range(nb, 0, -1):
            @pl.when(d <= n_my)
            def _():
                def go_w(qq):
                    scat_at(qq, qq % nb).wait()
                switch((n_my - d) % pnb, go_w)

        plsc.subcore_barrier()

        for m in range(rows_per_tile // _CH):
            off = s * rows_per_tile + m * _CH
            pltpu.sync_copy(acc.at[pl.ds(off, _CH)],
                            out_hbm.at[c, pl.ds(off, _CH)])
        if with_deg:
            pltpu.sync_copy(deg_v, deg_hbm.at[c, s])

    return sc_agg


def _mean_h(x_ref, aa_ref, ab_ref, deg_ref, ws_ref, wn_ref, b_ref):
    deg = jnp.sum(deg_ref[...], axis=0)
    inv = (1.0 / jnp.maximum(deg, 1.0))[:, None]
    mean = (aa_ref[...] + ab_ref[...]) * inv
    h = (jnp.dot(x_ref[...], ws_ref[...], preferred_element_type=jnp.float32)
         + jnp.dot(mean, wn_ref[...], preferred_element_type=jnp.float32)
         + b_ref[...])
    return jnp.maximum(h, 0.0)


def _dense1_body(x_ref, aa_ref, ab_ref, deg_ref, ws_ref, wn_ref, b_ref,
                 o_ref):
    o_ref[...] = _mean_h(x_ref, aa_ref, ab_ref, deg_ref, ws_ref, wn_ref,
                         b_ref)


def _dense2_body(x_ref, aa_ref, ab_ref, deg_ref, ws_ref, wn_ref, b_ref,
                 wc_ref, bc_ref, o_ref):
    h = _mean_h(x_ref, aa_ref, ab_ref, deg_ref, ws_ref, wn_ref, b_ref)
    o_ref[...] = (jnp.dot(h, wc_ref[...], preferred_element_type=jnp.float32)
                  + bc_ref[...])


def _dense(body, n_pad, nw, d, h, extra_w, r=1024):
    grid = n_pad // r
    row_spec = pl.BlockSpec((r, d), lambda i: (i, 0))
    full = lambda shape: pl.BlockSpec(shape, lambda i: (0,) * len(shape))
    in_specs = [row_spec, row_spec, row_spec,
                pl.BlockSpec((nw, r), lambda i: (0, i)),
                full((d, h)), full((d, h)), full((1, h))]
    out_d = h
    for w in extra_w:
        in_specs += [full(w[0]), full(w[1])]
        out_d = w[0][1]
    return pl.pallas_call(
        body,
        grid=(grid,),
        in_specs=in_specs,
        out_specs=pl.BlockSpec((r, out_d), lambda i: (i, 0)),
        out_shape=jax.ShapeDtypeStruct((n_pad, out_d), jnp.float32),
    )


def kernel(x, edge_index, W1_self, W1_neigh, b1, W2_self, W2_neigh, b2,
           Wc, bc):
    n, d = x.shape
    h_dim = W1_self.shape[1]
    o_dim = W2_self.shape[1]
    c_dim = Wc.shape[1]
    e = edge_index.shape[1]

    n_pad = -(-n // 2048) * 2048
    e_pad = -(-e // _CH) * _CH

    src = edge_index[0].astype(jnp.int32)
    dst = edge_index[1].astype(jnp.int32)
    pad = e_pad - e
    src = jnp.concatenate([src, jnp.zeros((pad + _CH,), jnp.int32)])
    dst = jnp.concatenate(
        [dst, jnp.full((pad + _CH,), n_pad - 1, jnp.int32)])

    agg1, degp = _make_sc_agg(n_pad, d, e_pad, True)(x, src, dst)
    nw = degp.shape[0] * degp.shape[1]
    degp = degp.reshape(nw, n_pad)

    x_pad = jnp.zeros((n_pad, d), jnp.float32).at[:n].set(x)
    h1 = _dense(_dense1_body, n_pad, nw, d, h_dim, [])(
        x_pad, agg1[0], agg1[1], degp, W1_self, W1_neigh,
        b1.reshape(1, h_dim))

    (agg2,) = _make_sc_agg(n_pad, h_dim, e_pad, False)(h1, src, dst)

    out = _dense(_dense2_body, n_pad, nw, h_dim, o_dim,
                 [((o_dim, c_dim), (1, c_dim))])(
        h1, agg2[0], agg2[1], degp, W2_self, W2_neigh,
        b2.reshape(1, o_dim), Wc, bc.reshape(1, c_dim))
    return out[:n]


# speedup vs baseline: 12.8721x; 1.0433x over previous
import functools

import jax
import jax.numpy as jnp
from jax import lax
from jax.experimental import pallas as pl
from jax.experimental.pallas import tpu as pltpu
from jax.experimental.pallas import tpu_sc as plsc

_CH = 64


def _make_sc_agg(n_pad, d_row, n_edges, with_deg):
    assert n_edges % _CH == 0
    n_chunks = n_edges // _CH
    mesh = plsc.VectorSubcoreMesh(core_axis_name="c", subcore_axis_name="s")
    nc, ns = mesh.num_cores, mesh.num_subcores
    nw = nc * ns
    rows_per_tile = n_pad // ns
    assert rows_per_tile % _CH == 0
    n_base, n_rem = n_chunks // nw, n_chunks % nw
    nb = 4
    la = 2
    pnb = 8
    pd = pnb - nb

    out_type = [jax.ShapeDtypeStruct((nc, n_pad, d_row), jnp.float32)]
    scratch = [
        *[pltpu.VMEM((_CH,), jnp.int32) for _ in range(pnb)],
        *[pltpu.VMEM((_CH,), jnp.int32) for _ in range(pnb)],
        *[pltpu.VMEM((_CH, d_row), jnp.float32) for _ in range(nb)],
        pltpu.VMEM_SHARED((n_pad, d_row), jnp.float32),
        *[pltpu.SemaphoreType.DMA for _ in range(2 * pnb + 2 * nb)],
    ]
    if with_deg:
        out_type.append(jax.ShapeDtypeStruct((nc, ns, n_pad), jnp.float32))
        scratch.append(pltpu.VMEM((n_pad,), jnp.float32))

    @functools.partial(
        pl.kernel, out_type=out_type, mesh=mesh, scratch_types=scratch,
        compiler_params=pltpu.CompilerParams(needs_layout_passes=False))
    def sc_agg(feat_hbm, src_hbm, dst_hbm, out_hbm, *rest):
        if with_deg:
            deg_hbm, *rest = rest
            deg_v = rest[-1]
            rest = rest[:-1]
        src_bufs = rest[:pnb]
        dst_bufs = rest[pnb:2 * pnb]
        rows = rest[2 * pnb:2 * pnb + nb]
        acc = rest[2 * pnb + nb]
        sems = rest[2 * pnb + nb + 1:]
        sem_j, sem_i, sem_g, sem_s = (sems[:pnb], sems[pnb:2 * pnb],
                                      sems[2 * pnb:2 * pnb + nb],
                                      sems[2 * pnb + nb:])
        c = lax.axis_index("c")
        s = lax.axis_index("s")

        def zero_row(i, _):
            for k in range(d_row // 16):
                rows[0][i, pl.ds(k * 16, 16)] = jnp.zeros((16,), jnp.float32)
            return 0

        lax.fori_loop(0, _CH, zero_row, 0)
        for m in range(rows_per_tile // _CH):
            pltpu.sync_copy(
                rows[0], acc.at[pl.ds(s * rows_per_tile + m * _CH, _CH)])
        if with_deg:
            def zero_deg(i, _):
                deg_v[pl.ds(i * 16, 16)] = jnp.zeros((16,), jnp.float32)
                return 0
            lax.fori_loop(0, n_pad // 16, zero_deg, 0)
        plsc.subcore_barrier()

        wid = s * nc + c
        n_my = n_base + jnp.where(wid < n_rem, 1, 0)
        start = wid * n_base + jnp.minimum(wid, n_rem)
        base_e = start * _CH

        def sidx_at(j, q):
            return pltpu.make_async_copy(
                src_hbm.at[pl.ds(base_e + j * _CH, _CH)], src_bufs[q],
                sem_j[q])

        def gather_at(q, b):
            return pltpu.make_async_copy(
                feat_hbm.at[src_bufs[q]], rows[b], sem_g[b])

        def idx_at(j, q):
            return pltpu.make_async_copy(
                dst_hbm.at[pl.ds(base_e + j * _CH, _CH)], dst_bufs[q],
                sem_i[q])

        def scat_at(q, b):
            return pltpu.make_async_copy(rows[b], acc.at[dst_bufs[q]],
                                         sem_s[b])

        def switch(q, fn):
            for qq in range(pnb):
                @pl.when(q == qq)
                def _():
                    fn(qq)

        for u in range(pd):
            @pl.when(u < n_my)
            def _():
                sidx_at(u, u).start()
                idx_at(u, u).start()

        def body(t, _):
            @pl.when(t < n_my)
            def _issue():
                @pl.when(t >= nb)
                def _():
                    def go_w(qq):
                        scat_at(qq, qq % nb).wait()
                    switch((t - nb) % pnb, go_w)

                u = t + pd

                @pl.when(u < n_my)
                def _prefetch():
                    def go_pf(qq):
                        sidx_at(u, qq).start()
                        idx_at(u, qq).start()
                    switch(u % pnb, go_pf)

                def go_g(qq):
                    sidx_at(t, qq).wait()
                    gather_at(qq, qq % nb).start()
                switch(t % pnb, go_g)

            @pl.when(t >= la)
            def _consume():
                j = t - la

                def go(qq):
                    idx_at(j, qq).wait()
                    gather_at(qq, qq % nb).wait()
                    scat_at(qq, qq % nb).start(add=True)
                    if with_deg:
                        for k in range(_CH // 16):
                            v = dst_bufs[qq][pl.ds(k * 16, 16)]
                            cnt, last = plsc.scan_count(v)
                            plsc.addupdate_scatter(
                                deg_v, [v], cnt.astype(jnp.float32),
                                mask=last)
                switch(j % pnb, go)
            return 0

        lax.fori_loop(0, n_my + la, body, 0)
        for d in range(nb, 0, -1):
            @pl.when(d <= n_my)
            def _():
                def go_w(qq):
                    scat_at(qq, qq % nb).wait()
                switch((n_my - d) % pnb, go_w)

        plsc.subcore_barrier()

        for m in range(rows_per_tile // _CH):
            off = s * rows_per_tile + m * _CH
            pltpu.sync_copy(acc.at[pl.ds(off, _CH)],
                            out_hbm.at[c, pl.ds(off, _CH)])
        if with_deg:
            pltpu.sync_copy(deg_v, deg_hbm.at[c, s])

    return sc_agg


def _mean_h(x_ref, aa_ref, ab_ref, deg_ref, ws_ref, wn_ref, b_ref):
    deg = jnp.sum(deg_ref[...], axis=0)
    inv = (1.0 / jnp.maximum(deg, 1.0))[:, None]
    mean = (aa_ref[...] + ab_ref[...]) * inv
    h = (jnp.dot(x_ref[...], ws_ref[...], preferred_element_type=jnp.float32)
         + jnp.dot(mean, wn_ref[...], preferred_element_type=jnp.float32)
         + b_ref[...])
    return jnp.maximum(h, 0.0)


def _dense1_body(x_ref, aa_ref, ab_ref, deg_ref, ws_ref, wn_ref, b_ref,
                 o_ref):
    o_ref[...] = _mean_h(x_ref, aa_ref, ab_ref, deg_ref, ws_ref, wn_ref,
                         b_ref)


def _dense2_body(x_ref, aa_ref, ab_ref, deg_ref, ws_ref, wn_ref, b_ref,
                 wc_ref, bc_ref, o_ref):
    h = _mean_h(x_ref, aa_ref, ab_ref, deg_ref, ws_ref, wn_ref, b_ref)
    o_ref[...] = (jnp.dot(h, wc_ref[...], preferred_element_type=jnp.float32)
                  + bc_ref[...])


def _dense(body, n_pad, nw, d, h, extra_w, r=1024):
    grid = n_pad // r
    row_spec = pl.BlockSpec((r, d), lambda i: (i, 0))
    full = lambda shape: pl.BlockSpec(shape, lambda i: (0,) * len(shape))
    in_specs = [row_spec, row_spec, row_spec,
                pl.BlockSpec((nw, r), lambda i: (0, i)),
                full((d, h)), full((d, h)), full((1, h))]
    out_d = h
    for w in extra_w:
        in_specs += [full(w[0]), full(w[1])]
        out_d = w[0][1]
    return pl.pallas_call(
        body,
        grid=(grid,),
        in_specs=in_specs,
        out_specs=pl.BlockSpec((r, out_d), lambda i: (i, 0)),
        out_shape=jax.ShapeDtypeStruct((n_pad, out_d), jnp.float32),
    )


def kernel(x, edge_index, W1_self, W1_neigh, b1, W2_self, W2_neigh, b2,
           Wc, bc):
    n, d = x.shape
    h_dim = W1_self.shape[1]
    o_dim = W2_self.shape[1]
    c_dim = Wc.shape[1]
    e = edge_index.shape[1]

    n_pad = -(-n // 2048) * 2048
    e_pad = -(-e // _CH) * _CH

    src = edge_index[0].astype(jnp.int32)
    dst = edge_index[1].astype(jnp.int32)
    pad = e_pad - e
    src = jnp.concatenate([src, jnp.zeros((pad + _CH,), jnp.int32)])
    dst = jnp.concatenate(
        [dst, jnp.full((pad + _CH,), n_pad - 1, jnp.int32)])

    agg1, degp = _make_sc_agg(n_pad, d, e_pad, True)(x, src, dst)
    nw = degp.shape[0] * degp.shape[1]
    degp = degp.reshape(nw, n_pad)

    x_pad = jnp.zeros((n_pad, d), jnp.float32).at[:n].set(x)
    h1 = _dense(_dense1_body, n_pad, nw, d, h_dim, [])(
        x_pad, agg1[0], agg1[1], degp, W1_self, W1_neigh,
        b1.reshape(1, h_dim))

    (agg2,) = _make_sc_agg(n_pad, h_dim, e_pad, False)(h1, src, dst)

    out = _dense(_dense2_body, n_pad, nw, h_dim, o_dim,
                 [((o_dim, c_dim), (1, c_dim))])(
        h1, agg2[0], agg2[1], degp, W2_self, W2_neigh,
        b2.reshape(1, o_dim), Wc, bc.reshape(1, c_dim))
    return out[:n]


# speedup vs baseline: 13.0759x; 1.0158x over previous
import functools

import jax
import jax.numpy as jnp
from jax import lax
from jax.experimental import pallas as pl
from jax.experimental.pallas import tpu as pltpu
from jax.experimental.pallas import tpu_sc as plsc

_CH = 64


def _make_sc_agg(n_pad, d_row, n_edges, with_deg):
    assert n_edges % _CH == 0
    n_chunks = n_edges // _CH
    mesh = plsc.VectorSubcoreMesh(core_axis_name="c", subcore_axis_name="s")
    nc, ns = mesh.num_cores, mesh.num_subcores
    nw = nc * ns
    rows_per_tile = n_pad // ns
    assert rows_per_tile % _CH == 0
    n_base, n_rem = n_chunks // nw, n_chunks % nw
    nb = 4
    la = 2
    pnb = 8
    pd = pnb - nb

    out_type = [jax.ShapeDtypeStruct((nc, n_pad, d_row), jnp.float32)]
    scratch = [
        *[pltpu.VMEM((_CH,), jnp.int32) for _ in range(pnb)],
        *[pltpu.VMEM((_CH,), jnp.int32) for _ in range(pnb)],
        *[pltpu.VMEM((_CH, d_row), jnp.float32) for _ in range(nb)],
        pltpu.VMEM_SHARED((n_pad, d_row), jnp.float32),
        *[pltpu.SemaphoreType.DMA
          for _ in range(2 * pnb + 2 * nb + min(4, rows_per_tile // _CH))],
    ]
    if with_deg:
        out_type.append(jax.ShapeDtypeStruct((nc, ns, n_pad), jnp.float32))
        scratch.append(pltpu.VMEM((n_pad,), jnp.float32))

    @functools.partial(
        pl.kernel, out_type=out_type, mesh=mesh, scratch_types=scratch,
        compiler_params=pltpu.CompilerParams(needs_layout_passes=False))
    def sc_agg(feat_hbm, src_hbm, dst_hbm, out_hbm, *rest):
        if with_deg:
            deg_hbm, *rest = rest
            deg_v = rest[-1]
            rest = rest[:-1]
        src_bufs = rest[:pnb]
        dst_bufs = rest[pnb:2 * pnb]
        rows = rest[2 * pnb:2 * pnb + nb]
        acc = rest[2 * pnb + nb]
        sems = rest[2 * pnb + nb + 1:]
        sem_j, sem_i, sem_g, sem_s, sem_t = (
            sems[:pnb], sems[pnb:2 * pnb], sems[2 * pnb:2 * pnb + nb],
            sems[2 * pnb + nb:2 * pnb + 2 * nb], sems[2 * pnb + 2 * nb:])
        c = lax.axis_index("c")
        s = lax.axis_index("s")

        wid = s * nc + c
        n_my = n_base + jnp.where(wid < n_rem, 1, 0)
        start = wid * n_base + jnp.minimum(wid, n_rem)
        base_e = start * _CH

        def sidx_at(j, q):
            return pltpu.make_async_copy(
                src_hbm.at[pl.ds(base_e + j * _CH, _CH)], src_bufs[q],
                sem_j[q])

        def gather_at(q, b):
            return pltpu.make_async_copy(
                feat_hbm.at[src_bufs[q]], rows[b], sem_g[b])

        def idx_at(j, q):
            return pltpu.make_async_copy(
                dst_hbm.at[pl.ds(base_e + j * _CH, _CH)], dst_bufs[q],
                sem_i[q])

        def scat_at(q, b):
            return pltpu.make_async_copy(rows[b], acc.at[dst_bufs[q]],
                                         sem_s[b])

        def switch(q, fn):
            for qq in range(pnb):
                @pl.when(q == qq)
                def _():
                    fn(qq)

        for u in range(pd):
            @pl.when(u < n_my)
            def _():
                sidx_at(u, u).start()
                idx_at(u, u).start()

        def zero_row(i, _):
            for k in range(d_row // 16):
                rows[0][i, pl.ds(k * 16, 16)] = jnp.zeros((16,), jnp.float32)
            return 0

        lax.fori_loop(0, _CH, zero_row, 0)
        nt = len(sem_t)
        zcps = [
            pltpu.make_async_copy(
                rows[0], acc.at[pl.ds(s * rows_per_tile + m * _CH, _CH)],
                sem_t[m % nt])
            for m in range(rows_per_tile // _CH)]
        for m, cp in enumerate(zcps):
            if m >= nt:
                zcps[m - nt].wait()
            cp.start()
        if with_deg:
            def zero_deg(i, _):
                deg_v[pl.ds(i * 16, 16)] = jnp.zeros((16,), jnp.float32)
                return 0
            lax.fori_loop(0, n_pad // 16, zero_deg, 0)
        for cp in zcps[-nt:]:
            cp.wait()
        plsc.subcore_barrier()

        def body(t, _):
            @pl.when(t < n_my)
            def _issue():
                @pl.when(t >= nb)
                def _():
                    def go_w(qq):
                        scat_at(qq, qq % nb).wait()
                    switch((t - nb) % pnb, go_w)

                u = t + pd

                @pl.when(u < n_my)
                def _prefetch():
                    def go_pf(qq):
                        sidx_at(u, qq).start()
                        idx_at(u, qq).start()
                    switch(u % pnb, go_pf)

                def go_g(qq):
                    sidx_at(t, qq).wait()
                    gather_at(qq, qq % nb).start()
                switch(t % pnb, go_g)

            @pl.when(t >= la)
            def _consume():
                j = t - la

                def go(qq):
                    idx_at(j, qq).wait()
                    gather_at(qq, qq % nb).wait()
                    scat_at(qq, qq % nb).start(add=True)
                    if with_deg:
                        for k in range(_CH // 16):
                            v = dst_bufs[qq][pl.ds(k * 16, 16)]
                            cnt, last = plsc.scan_count(v)
                            plsc.addupdate_scatter(
                                deg_v, [v], cnt.astype(jnp.float32),
                                mask=last)
                switch(j % pnb, go)
            return 0

        lax.fori_loop(0, n_my + la, body, 0)
        for d in range(nb, 0, -1):
            @pl.when(d <= n_my)
            def _():
                def go_w(qq):
                    scat_at(qq, qq % nb).wait()
                switch((n_my - d) % pnb, go_w)

        plsc.subcore_barrier()

        ocps = []
        for m in range(rows_per_tile // _CH):
            off = s * rows_per_tile + m * _CH
            ocps.append(pltpu.make_async_copy(
                acc.at[pl.ds(off, _CH)], out_hbm.at[c, pl.ds(off, _CH)],
                sem_t[m % nt]))
        for m, cp in enumerate(ocps):
            if m >= nt:
                ocps[m - nt].wait()
            cp.start()
        if with_deg:
            pltpu.sync_copy(deg_v, deg_hbm.at[c, s])
        for cp in ocps[-nt:]:
            cp.wait()

    return sc_agg


def _mean_h(x_ref, aa_ref, ab_ref, deg_ref, ws_ref, wn_ref, b_ref):
    deg = jnp.sum(deg_ref[...], axis=0)
    inv = (1.0 / jnp.maximum(deg, 1.0))[:, None]
    mean = (aa_ref[...] + ab_ref[...]) * inv
    h = (jnp.dot(x_ref[...], ws_ref[...], preferred_element_type=jnp.float32)
         + jnp.dot(mean, wn_ref[...], preferred_element_type=jnp.float32)
         + b_ref[...])
    return jnp.maximum(h, 0.0)


def _dense1_body(x_ref, aa_ref, ab_ref, deg_ref, ws_ref, wn_ref, b_ref,
                 o_ref):
    o_ref[...] = _mean_h(x_ref, aa_ref, ab_ref, deg_ref, ws_ref, wn_ref,
                         b_ref)


def _dense2_body(x_ref, aa_ref, ab_ref, deg_ref, ws_ref, wn_ref, b_ref,
                 wc_ref, bc_ref, o_ref):
    h = _mean_h(x_ref, aa_ref, ab_ref, deg_ref, ws_ref, wn_ref, b_ref)
    o_ref[...] = (jnp.dot(h, wc_ref[...], preferred_element_type=jnp.float32)
                  + bc_ref[...])


def _dense(body, n_pad, nw, d, h, extra_w, r=1024):
    grid = n_pad // r
    row_spec = pl.BlockSpec((r, d), lambda i: (i, 0))
    full = lambda shape: pl.BlockSpec(shape, lambda i: (0,) * len(shape))
    in_specs = [row_spec, row_spec, row_spec,
                pl.BlockSpec((nw, r), lambda i: (0, i)),
                full((d, h)), full((d, h)), full((1, h))]
    out_d = h
    for w in extra_w:
        in_specs += [full(w[0]), full(w[1])]
        out_d = w[0][1]
    return pl.pallas_call(
        body,
        grid=(grid,),
        in_specs=in_specs,
        out_specs=pl.BlockSpec((r, out_d), lambda i: (i, 0)),
        out_shape=jax.ShapeDtypeStruct((n_pad, out_d), jnp.float32),
    )


def kernel(x, edge_index, W1_self, W1_neigh, b1, W2_self, W2_neigh, b2,
           Wc, bc):
    n, d = x.shape
    h_dim = W1_self.shape[1]
    o_dim = W2_self.shape[1]
    c_dim = Wc.shape[1]
    e = edge_index.shape[1]

    n_pad = -(-n // 2048) * 2048
    e_pad = -(-e // _CH) * _CH

    src = edge_index[0].astype(jnp.int32)
    dst = edge_index[1].astype(jnp.int32)
    pad = e_pad - e
    src = jnp.concatenate([src, jnp.zeros((pad + _CH,), jnp.int32)])
    dst = jnp.concatenate(
        [dst, jnp.full((pad + _CH,), n_pad - 1, jnp.int32)])

    agg1, degp = _make_sc_agg(n_pad, d, e_pad, True)(x, src, dst)
    nw = degp.shape[0] * degp.shape[1]
    degp = degp.reshape(nw, n_pad)

    x_pad = jnp.zeros((n_pad, d), jnp.float32).at[:n].set(x)
    h1 = _dense(_dense1_body, n_pad, nw, d, h_dim, [])(
        x_pad, agg1[0], agg1[1], degp, W1_self, W1_neigh,
        b1.reshape(1, h_dim))

    (agg2,) = _make_sc_agg(n_pad, h_dim, e_pad, False)(h1, src, dst)

    out = _dense(_dense2_body, n_pad, nw, h_dim, o_dim,
                 [((o_dim, c_dim), (1, c_dim))])(
        h1, agg2[0], agg2[1], degp, W2_self, W2_neigh,
        b2.reshape(1, o_dim), Wc, bc.reshape(1, c_dim))
    return out[:n]


# speedup vs baseline: 13.1854x; 1.0084x over previous
import functools

import jax
import jax.numpy as jnp
from jax import lax
from jax.experimental import pallas as pl
from jax.experimental.pallas import tpu as pltpu
from jax.experimental.pallas import tpu_sc as plsc

_CH = 64


def _make_sc_agg(n_pad, d_row, n_edges, with_deg):
    assert n_edges % _CH == 0
    n_chunks = n_edges // _CH
    mesh = plsc.VectorSubcoreMesh(core_axis_name="c", subcore_axis_name="s")
    nc, ns = mesh.num_cores, mesh.num_subcores
    nw = nc * ns
    rows_per_tile = n_pad // ns
    assert rows_per_tile % _CH == 0
    n_base, n_rem = n_chunks // nw, n_chunks % nw
    nb = 4
    la = 3
    pnb = 8
    pd = pnb - nb

    out_type = [jax.ShapeDtypeStruct((nc, n_pad, d_row), jnp.float32)]
    scratch = [
        *[pltpu.VMEM((_CH,), jnp.int32) for _ in range(pnb)],
        *[pltpu.VMEM((_CH,), jnp.int32) for _ in range(pnb)],
        *[pltpu.VMEM((_CH, d_row), jnp.float32) for _ in range(nb)],
        pltpu.VMEM_SHARED((n_pad, d_row), jnp.float32),
        *[pltpu.SemaphoreType.DMA
          for _ in range(2 * pnb + 2 * nb + min(4, rows_per_tile // _CH))],
    ]
    if with_deg:
        out_type.append(jax.ShapeDtypeStruct((nc, ns, n_pad), jnp.float32))
        scratch.append(pltpu.VMEM((n_pad,), jnp.float32))

    @functools.partial(
        pl.kernel, out_type=out_type, mesh=mesh, scratch_types=scratch,
        compiler_params=pltpu.CompilerParams(needs_layout_passes=False))
    def sc_agg(feat_hbm, src_hbm, dst_hbm, out_hbm, *rest):
        if with_deg:
            deg_hbm, *rest = rest
            deg_v = rest[-1]
            rest = rest[:-1]
        src_bufs = rest[:pnb]
        dst_bufs = rest[pnb:2 * pnb]
        rows = rest[2 * pnb:2 * pnb + nb]
        acc = rest[2 * pnb + nb]
        sems = rest[2 * pnb + nb + 1:]
        sem_j, sem_i, sem_g, sem_s, sem_t = (
            sems[:pnb], sems[pnb:2 * pnb], sems[2 * pnb:2 * pnb + nb],
            sems[2 * pnb + nb:2 * pnb + 2 * nb], sems[2 * pnb + 2 * nb:])
        c = lax.axis_index("c")
        s = lax.axis_index("s")

        wid = s * nc + c
        n_my = n_base + jnp.where(wid < n_rem, 1, 0)
        start = wid * n_base + jnp.minimum(wid, n_rem)
        base_e = start * _CH

        def sidx_at(j, q):
            return pltpu.make_async_copy(
                src_hbm.at[pl.ds(base_e + j * _CH, _CH)], src_bufs[q],
                sem_j[q])

        def gather_at(q, b):
            return pltpu.make_async_copy(
                feat_hbm.at[src_bufs[q]], rows[b], sem_g[b])

        def idx_at(j, q):
            return pltpu.make_async_copy(
                dst_hbm.at[pl.ds(base_e + j * _CH, _CH)], dst_bufs[q],
                sem_i[q])

        def scat_at(q, b):
            return pltpu.make_async_copy(rows[b], acc.at[dst_bufs[q]],
                                         sem_s[b])

        def switch(q, fn):
            for qq in range(pnb):
                @pl.when(q == qq)
                def _():
                    fn(qq)

        for u in range(pd):
            @pl.when(u < n_my)
            def _():
                sidx_at(u, u).start()
                idx_at(u, u).start()

        def zero_row(i, _):
            for k in range(d_row // 16):
                rows[0][i, pl.ds(k * 16, 16)] = jnp.zeros((16,), jnp.float32)
            return 0

        lax.fori_loop(0, _CH, zero_row, 0)
        nt = len(sem_t)
        zcps = [
            pltpu.make_async_copy(
                rows[0], acc.at[pl.ds(s * rows_per_tile + m * _CH, _CH)],
                sem_t[m % nt])
            for m in range(rows_per_tile // _CH)]
        for m, cp in enumerate(zcps):
            if m >= nt:
                zcps[m - nt].wait()
            cp.start()
        if with_deg:
            def zero_deg(i, _):
                deg_v[pl.ds(i * 16, 16)] = jnp.zeros((16,), jnp.float32)
                return 0
            lax.fori_loop(0, n_pad // 16, zero_deg, 0)
        for cp in zcps[-nt:]:
            cp.wait()
        plsc.subcore_barrier()

        def body(t, _):
            @pl.when(t < n_my)
            def _issue():
                @pl.when(t >= nb)
                def _():
                    def go_w(qq):
                        scat_at(qq, qq % nb).wait()
                    switch((t - nb) % pnb, go_w)

                u = t + pd

                @pl.when(u < n_my)
                def _prefetch():
                    def go_pf(qq):
                        sidx_at(u, qq).start()
                        idx_at(u, qq).start()
                    switch(u % pnb, go_pf)

                def go_g(qq):
                    sidx_at(t, qq).wait()
                    gather_at(qq, qq % nb).start()
                switch(t % pnb, go_g)

            @pl.when(t >= la)
            def _consume():
                j = t - la

                def go(qq):
                    idx_at(j, qq).wait()
                    gather_at(qq, qq % nb).wait()
                    scat_at(qq, qq % nb).start(add=True)
                    if with_deg:
                        for k in range(_CH // 16):
                            v = dst_bufs[qq][pl.ds(k * 16, 16)]
                            cnt, last = plsc.scan_count(v)
                            plsc.addupdate_scatter(
                                deg_v, [v], cnt.astype(jnp.float32),
                                mask=last)
                switch(j % pnb, go)
            return 0

        lax.fori_loop(0, n_my + la, body, 0)
        for d in range(nb, 0, -1):
            @pl.when(d <= n_my)
            def _():
                def go_w(qq):
                    scat_at(qq, qq % nb).wait()
                switch((n_my - d) % pnb, go_w)

        plsc.subcore_barrier()

        ocps = []
        for m in range(rows_per_tile // _CH):
            off = s * rows_per_tile + m * _CH
            ocps.append(pltpu.make_async_copy(
                acc.at[pl.ds(off, _CH)], out_hbm.at[c, pl.ds(off, _CH)],
                sem_t[m % nt]))
        for m, cp in enumerate(ocps):
            if m >= nt:
                ocps[m - nt].wait()
            cp.start()
        if with_deg:
            pltpu.sync_copy(deg_v, deg_hbm.at[c, s])
        for cp in ocps[-nt:]:
            cp.wait()

    return sc_agg


def _mean_h(x_ref, aa_ref, ab_ref, deg_ref, ws_ref, wn_ref, b_ref):
    inv = 1.0 / jnp.maximum(deg_ref[...], 1.0)
    mean = (aa_ref[...] + ab_ref[...]) * inv
    h = (jnp.dot(x_ref[...], ws_ref[...], preferred_element_type=jnp.float32)
         + jnp.dot(mean, wn_ref[...], preferred_element_type=jnp.float32)
         + b_ref[...])
    return jnp.maximum(h, 0.0)


def _dense1_body(x_ref, aa_ref, ab_ref, deg_ref, ws_ref, wn_ref, b_ref,
                 o_ref):
    o_ref[...] = _mean_h(x_ref, aa_ref, ab_ref, deg_ref, ws_ref, wn_ref,
                         b_ref)


def _dense2_body(x_ref, aa_ref, ab_ref, deg_ref, ws_ref, wn_ref, b_ref,
                 wc_ref, bc_ref, o_ref):
    h = _mean_h(x_ref, aa_ref, ab_ref, deg_ref, ws_ref, wn_ref, b_ref)
    o_ref[...] = (jnp.dot(h, wc_ref[...], preferred_element_type=jnp.float32)
                  + bc_ref[...])


def _dense(body, n_rows, d, h, extra_w, r=1000):
    grid = n_rows // r
    assert n_rows % r == 0
    row_spec = pl.BlockSpec((r, d), lambda i: (i, 0))
    full = lambda shape: pl.BlockSpec(shape, lambda i: (0,) * len(shape))
    in_specs = [row_spec, row_spec, row_spec,
                pl.BlockSpec((r, 1), lambda i: (i, 0)),
                full((d, h)), full((d, h)), full((1, h))]
    out_d = h
    for w in extra_w:
        in_specs += [full(w[0]), full(w[1])]
        out_d = w[0][1]
    return pl.pallas_call(
        body,
        grid=(grid,),
        in_specs=in_specs,
        out_specs=pl.BlockSpec((r, out_d), lambda i: (i, 0)),
        out_shape=jax.ShapeDtypeStruct((n_rows, out_d), jnp.float32),
    )


def kernel(x, edge_index, W1_self, W1_neigh, b1, W2_self, W2_neigh, b2,
           Wc, bc):
    n, d = x.shape
    h_dim = W1_self.shape[1]
    o_dim = W2_self.shape[1]
    c_dim = Wc.shape[1]
    e = edge_index.shape[1]

    n_pad = -(-n // 2048) * 2048
    e_pad = -(-e // _CH) * _CH

    src = edge_index[0].astype(jnp.int32)
    dst = edge_index[1].astype(jnp.int32)
    pad = e_pad - e
    src = jnp.concatenate([src, jnp.zeros((pad + _CH,), jnp.int32)])
    dst = jnp.concatenate(
        [dst, jnp.full((pad + _CH,), n_pad - 1, jnp.int32)])

    agg1, degp = _make_sc_agg(n_pad, d, e_pad, True)(x, src, dst)
    deg = degp.reshape(-1, n_pad).sum(axis=0).reshape(n_pad, 1)

    h1 = _dense(_dense1_body, n, d, h_dim, [])(
        x, agg1[0], agg1[1], deg, W1_self, W1_neigh,
        b1.reshape(1, h_dim))

    (agg2,) = _make_sc_agg(n_pad, h_dim, e_pad, False)(h1, src, dst)

    out = _dense(_dense2_body, n, h_dim, o_dim,
                 [((o_dim, c_dim), (1, c_dim))])(
        h1, agg2[0], agg2[1], deg, W2_self, W2_neigh,
        b2.reshape(1, o_dim), Wc, bc.reshape(1, c_dim))
    return out


# speedup vs baseline: 13.5079x; 1.0245x over previous
import functools

import jax
import jax.numpy as jnp
from jax import lax
from jax.experimental import pallas as pl
from jax.experimental.pallas import tpu as pltpu
from jax.experimental.pallas import tpu_sc as plsc

_CH = 64


def _make_sc_agg(n_pad, d_row, n_edges, with_deg):
    assert n_edges % _CH == 0
    n_chunks = n_edges // _CH
    mesh = plsc.VectorSubcoreMesh(core_axis_name="c", subcore_axis_name="s")
    nc, ns = mesh.num_cores, mesh.num_subcores
    nw = nc * ns
    rows_per_tile = n_pad // ns
    assert rows_per_tile % _CH == 0
    n_base, n_rem = n_chunks // nw, n_chunks % nw
    nb = 4
    la = 3
    pnb = 8
    pd = pnb - nb

    out_type = [jax.ShapeDtypeStruct((nc, n_pad, d_row), jnp.float32)]
    scratch = [
        *[pltpu.VMEM((_CH,), jnp.int32) for _ in range(pnb)],
        *[pltpu.VMEM((_CH,), jnp.int32) for _ in range(pnb)],
        *[pltpu.VMEM((_CH, d_row), jnp.float32) for _ in range(nb)],
        pltpu.VMEM_SHARED((n_pad, d_row), jnp.float32),
        *[pltpu.SemaphoreType.DMA
          for _ in range(2 * pnb + 2 * nb + min(4, rows_per_tile // _CH))],
    ]
    if with_deg:
        out_type.append(jax.ShapeDtypeStruct((nc, ns, n_pad), jnp.float32))
        scratch.append(pltpu.VMEM((n_pad,), jnp.float32))

    @functools.partial(
        pl.kernel, out_type=out_type, mesh=mesh, scratch_types=scratch,
        compiler_params=pltpu.CompilerParams(needs_layout_passes=False))
    def sc_agg(feat_hbm, src_hbm, dst_hbm, out_hbm, *rest):
        if with_deg:
            deg_hbm, *rest = rest
            deg_v = rest[-1]
            rest = rest[:-1]
        src_bufs = rest[:pnb]
        dst_bufs = rest[pnb:2 * pnb]
        rows = rest[2 * pnb:2 * pnb + nb]
        acc = rest[2 * pnb + nb]
        sems = rest[2 * pnb + nb + 1:]
        sem_j, sem_i, sem_g, sem_s, sem_t = (
            sems[:pnb], sems[pnb:2 * pnb], sems[2 * pnb:2 * pnb + nb],
            sems[2 * pnb + nb:2 * pnb + 2 * nb], sems[2 * pnb + 2 * nb:])
        c = lax.axis_index("c")
        s = lax.axis_index("s")

        wid = s * nc + c
        n_my = n_base + jnp.where(wid < n_rem, 1, 0)
        start = wid * n_base + jnp.minimum(wid, n_rem)
        base_e = start * _CH

        def sidx_at(j, q):
            return pltpu.make_async_copy(
                src_hbm.at[pl.ds(base_e + j * _CH, _CH)], src_bufs[q],
                sem_j[q])

        def gather_at(q, b):
            return pltpu.make_async_copy(
                feat_hbm.at[src_bufs[q]], rows[b], sem_g[b])

        def idx_at(j, q):
            return pltpu.make_async_copy(
                dst_hbm.at[pl.ds(base_e + j * _CH, _CH)], dst_bufs[q],
                sem_i[q])

        def scat_at(q, b):
            return pltpu.make_async_copy(rows[b], acc.at[dst_bufs[q]],
                                         sem_s[b])

        def switch(q, fn):
            for qq in range(pnb):
                @pl.when(q == qq)
                def _():
                    fn(qq)

        for u in range(pd):
            @pl.when(u < n_my)
            def _():
                sidx_at(u, u).start()
                idx_at(u, u).start()

        def zero_row(i, _):
            for k in range(d_row // 16):
                rows[0][i, pl.ds(k * 16, 16)] = jnp.zeros((16,), jnp.float32)
            return 0

        lax.fori_loop(0, _CH, zero_row, 0)
        nt = len(sem_t)
        zcps = [
            pltpu.make_async_copy(
                rows[0], acc.at[pl.ds(s * rows_per_tile + m * _CH, _CH)],
                sem_t[m % nt])
            for m in range(rows_per_tile // _CH)]
        for m, cp in enumerate(zcps):
            if m >= nt:
                zcps[m - nt].wait()
            cp.start()
        if with_deg:
            def zero_deg(i, _):
                deg_v[pl.ds(i * 16, 16)] = jnp.zeros((16,), jnp.float32)
                return 0
            lax.fori_loop(0, n_pad // 16, zero_deg, 0)
        for cp in zcps[-nt:]:
            cp.wait()
        plsc.subcore_barrier()

        def body(t, _):
            @pl.when(t < n_my)
            def _issue():
                @pl.when(t >= nb)
                def _():
                    def go_w(qq):
                        scat_at(qq, qq % nb).wait()
                    switch((t - nb) % pnb, go_w)

                u = t + pd

                @pl.when(u < n_my)
                def _prefetch():
                    def go_pf(qq):
                        sidx_at(u, qq).start()
                        idx_at(u, qq).start()
                    switch(u % pnb, go_pf)

                def go_g(qq):
                    sidx_at(t, qq).wait()
                    gather_at(qq, qq % nb).start()
                switch(t % pnb, go_g)

            @pl.when(t >= la)
            def _consume():
                j = t - la

                def go(qq):
                    idx_at(j, qq).wait()
                    gather_at(qq, qq % nb).wait()
                    scat_at(qq, qq % nb).start(add=True)
                    if with_deg:
                        for k in range(_CH // 16):
                            v = dst_bufs[qq][pl.ds(k * 16, 16)]
                            cnt, last = plsc.scan_count(v)
                            plsc.addupdate_scatter(
                                deg_v, [v], cnt.astype(jnp.float32),
                                mask=last)
                switch(j % pnb, go)
            return 0

        lax.fori_loop(0, n_my + la, body, 0)
        for d in range(nb, 0, -1):
            @pl.when(d <= n_my)
            def _():
                def go_w(qq):
                    scat_at(qq, qq % nb).wait()
                switch((n_my - d) % pnb, go_w)

        plsc.subcore_barrier()

        if with_deg:
            pltpu.sync_copy(deg_v, deg_hbm.at[c, s])

        ocps = []
        for m in range(rows_per_tile // _CH):
            off = s * rows_per_tile + m * _CH
            ocps.append(pltpu.make_async_copy(
                acc.at[pl.ds(off, _CH)], out_hbm.at[c, pl.ds(off, _CH)],
                sem_t[m % nt]))
        for m, cp in enumerate(ocps):
            if m >= nt:
                ocps[m - nt].wait()
            cp.start()
        for cp in ocps[-nt:]:
            cp.wait()

    return sc_agg


def _mean_h(x_ref, aa_ref, ab_ref, deg_ref, ws_ref, wn_ref, b_ref):
    inv = 1.0 / jnp.maximum(deg_ref[...], 1.0)
    mean = (aa_ref[...] + ab_ref[...]) * inv
    h = (jnp.dot(x_ref[...], ws_ref[...], preferred_element_type=jnp.float32)
         + jnp.dot(mean, wn_ref[...], preferred_element_type=jnp.float32)
         + b_ref[...])
    return jnp.maximum(h, 0.0)


def _dense1_body(x_ref, aa_ref, ab_ref, deg_ref, ws_ref, wn_ref, b_ref,
                 o_ref):
    o_ref[...] = _mean_h(x_ref, aa_ref, ab_ref, deg_ref, ws_ref, wn_ref,
                         b_ref)


def _dense2_body(x_ref, aa_ref, ab_ref, deg_ref, ws_ref, wn_ref, b_ref,
                 wc_ref, bc_ref, o_ref):
    h = _mean_h(x_ref, aa_ref, ab_ref, deg_ref, ws_ref, wn_ref, b_ref)
    o_ref[...] = (jnp.dot(h, wc_ref[...], preferred_element_type=jnp.float32)
                  + bc_ref[...])


def _dense(body, n_rows, d, h, extra_w, r=2000):
    grid = n_rows // r
    assert n_rows % r == 0
    row_spec = pl.BlockSpec((r, d), lambda i: (i, 0))
    full = lambda shape: pl.BlockSpec(shape, lambda i: (0,) * len(shape))
    in_specs = [row_spec, row_spec, row_spec,
                pl.BlockSpec((r, 1), lambda i: (i, 0)),
                full((d, h)), full((d, h)), full((1, h))]
    out_d = h
    for w in extra_w:
        in_specs += [full(w[0]), full(w[1])]
        out_d = w[0][1]
    return pl.pallas_call(
        body,
        grid=(grid,),
        in_specs=in_specs,
        out_specs=pl.BlockSpec((r, out_d), lambda i: (i, 0)),
        out_shape=jax.ShapeDtypeStruct((n_rows, out_d), jnp.float32),
    )


def kernel(x, edge_index, W1_self, W1_neigh, b1, W2_self, W2_neigh, b2,
           Wc, bc):
    n, d = x.shape
    h_dim = W1_self.shape[1]
    o_dim = W2_self.shape[1]
    c_dim = Wc.shape[1]
    e = edge_index.shape[1]

    n_pad = -(-n // 2048) * 2048
    assert e % _CH == 0
    src = edge_index[0].astype(jnp.int32)
    dst = edge_index[1].astype(jnp.int32)

    agg1, degp = _make_sc_agg(n_pad, d, e, True)(x, src, dst)
    deg = degp.reshape(-1, n_pad).sum(axis=0).reshape(n_pad, 1)

    h1 = _dense(_dense1_body, n, d, h_dim, [])(
        x, agg1[0], agg1[1], deg, W1_self, W1_neigh,
        b1.reshape(1, h_dim))

    (agg2,) = _make_sc_agg(n_pad, h_dim, e, False)(h1, src, dst)

    out = _dense(_dense2_body, n, h_dim, o_dim,
                 [((o_dim, c_dim), (1, c_dim))])(
        h1, agg2[0], agg2[1], deg, W2_self, W2_neigh,
        b2.reshape(1, o_dim), Wc, bc.reshape(1, c_dim))
    return out


# speedup vs baseline: 14.9397x; 1.1060x over previous
import functools

import jax
import jax.numpy as jnp
from jax import lax
from jax.experimental import pallas as pl
from jax.experimental.pallas import tpu as pltpu
from jax.experimental.pallas import tpu_sc as plsc

_CH = 64


def _make_sc_agg(n_pad, d_row, n_edges, with_deg):
    assert n_edges % _CH == 0
    n_chunks = n_edges // _CH
    mesh = plsc.VectorSubcoreMesh(core_axis_name="c", subcore_axis_name="s")
    nc, ns = mesh.num_cores, mesh.num_subcores
    nw = nc * ns
    rows_per_tile = n_pad // ns
    assert rows_per_tile % _CH == 0
    n_base, n_rem = n_chunks // nw, n_chunks % nw
    nb = 4
    la = 3
    pnb = 8
    pd = pnb - nb

    out_type = [jax.ShapeDtypeStruct((nc, n_pad, d_row), jnp.float32)]
    scratch = [
        *[pltpu.VMEM((_CH,), jnp.int32) for _ in range(pnb)],
        *[pltpu.VMEM((_CH,), jnp.int32) for _ in range(pnb)],
        *[pltpu.VMEM((_CH, d_row), jnp.float32) for _ in range(nb)],
        pltpu.VMEM_SHARED((n_pad, d_row), jnp.float32),
        *[pltpu.SemaphoreType.DMA
          for _ in range(2 * pnb + 2 * nb + min(4, rows_per_tile // _CH))],
    ]
    if with_deg:
        out_type.append(jax.ShapeDtypeStruct((nc, ns, n_pad), jnp.float32))
        scratch.append(pltpu.VMEM((n_pad,), jnp.float32))

    @functools.partial(
        pl.kernel, out_type=out_type, mesh=mesh, scratch_types=scratch,
        compiler_params=pltpu.CompilerParams(needs_layout_passes=False))
    def sc_agg(feat_hbm, edges_hbm, out_hbm, *rest):
        if with_deg:
            deg_hbm, *rest = rest
            deg_v = rest[-1]
            rest = rest[:-1]
        src_bufs = rest[:pnb]
        dst_bufs = rest[pnb:2 * pnb]
        rows = rest[2 * pnb:2 * pnb + nb]
        acc = rest[2 * pnb + nb]
        sems = rest[2 * pnb + nb + 1:]
        sem_j, sem_i, sem_g, sem_s, sem_t = (
            sems[:pnb], sems[pnb:2 * pnb], sems[2 * pnb:2 * pnb + nb],
            sems[2 * pnb + nb:2 * pnb + 2 * nb], sems[2 * pnb + 2 * nb:])
        c = lax.axis_index("c")
        s = lax.axis_index("s")

        wid = s * nc + c
        n_my = n_base + jnp.where(wid < n_rem, 1, 0)
        start = wid * n_base + jnp.minimum(wid, n_rem)
        base_e = start * _CH

        def sidx_at(j, q):
            return pltpu.make_async_copy(
                edges_hbm.at[0, pl.ds(base_e + j * _CH, _CH)], src_bufs[q],
                sem_j[q])

        def gather_at(q, b):
            return pltpu.make_async_copy(
                feat_hbm.at[src_bufs[q]], rows[b], sem_g[b])

        def idx_at(j, q):
            return pltpu.make_async_copy(
                edges_hbm.at[1, pl.ds(base_e + j * _CH, _CH)], dst_bufs[q],
                sem_i[q])

        def scat_at(q, b):
            return pltpu.make_async_copy(rows[b], acc.at[dst_bufs[q]],
                                         sem_s[b])

        def switch(q, fn):
            for qq in range(pnb):
                @pl.when(q == qq)
                def _():
                    fn(qq)

        for u in range(pd):
            @pl.when(u < n_my)
            def _():
                sidx_at(u, u).start()
                idx_at(u, u).start()

        def zero_row(i, _):
            for k in range(d_row // 16):
                rows[0][i, pl.ds(k * 16, 16)] = jnp.zeros((16,), jnp.float32)
            return 0

        lax.fori_loop(0, _CH, zero_row, 0)
        nt = len(sem_t)
        zcps = [
            pltpu.make_async_copy(
                rows[0], acc.at[pl.ds(s * rows_per_tile + m * _CH, _CH)],
                sem_t[m % nt])
            for m in range(rows_per_tile // _CH)]
        for m, cp in enumerate(zcps):
            if m >= nt:
                zcps[m - nt].wait()
            cp.start()
        if with_deg:
            def zero_deg(i, _):
                deg_v[pl.ds(i * 16, 16)] = jnp.zeros((16,), jnp.float32)
                return 0
            lax.fori_loop(0, n_pad // 16, zero_deg, 0)
        for cp in zcps[-nt:]:
            cp.wait()
        plsc.subcore_barrier()

        def body(t, _):
            @pl.when(t < n_my)
            def _issue():
                @pl.when(t >= nb)
                def _():
                    def go_w(qq):
                        scat_at(qq, qq % nb).wait()
                    switch((t - nb) % pnb, go_w)

                u = t + pd

                @pl.when(u < n_my)
                def _prefetch():
                    def go_pf(qq):
                        sidx_at(u, qq).start()
                        idx_at(u, qq).start()
                    switch(u % pnb, go_pf)

                def go_g(qq):
                    sidx_at(t, qq).wait()
                    gather_at(qq, qq % nb).start()
                switch(t % pnb, go_g)

            @pl.when(t >= la)
            def _consume():
                j = t - la

                def go(qq):
                    idx_at(j, qq).wait()
                    gather_at(qq, qq % nb).wait()
                    scat_at(qq, qq % nb).start(add=True)
                    if with_deg:
                        for k in range(_CH // 16):
                            v = dst_bufs[qq][pl.ds(k * 16, 16)]
                            cnt, last = plsc.scan_count(v)
                            plsc.addupdate_scatter(
                                deg_v, [v], cnt.astype(jnp.float32),
                                mask=last)
                switch(j % pnb, go)
            return 0

        lax.fori_loop(0, n_my + la, body, 0)
        for d in range(nb, 0, -1):
            @pl.when(d <= n_my)
            def _():
                def go_w(qq):
                    scat_at(qq, qq % nb).wait()
                switch((n_my - d) % pnb, go_w)

        plsc.subcore_barrier()

        if with_deg:
            pltpu.sync_copy(deg_v, deg_hbm.at[c, s])

        ocps = []
        for m in range(rows_per_tile // _CH):
            off = s * rows_per_tile + m * _CH
            ocps.append(pltpu.make_async_copy(
                acc.at[pl.ds(off, _CH)], out_hbm.at[c, pl.ds(off, _CH)],
                sem_t[m % nt]))
        for m, cp in enumerate(ocps):
            if m >= nt:
                ocps[m - nt].wait()
            cp.start()
        for cp in ocps[-nt:]:
            cp.wait()

    return sc_agg


def _mean_h(x_ref, aa_ref, ab_ref, deg_ref, ws_ref, wn_ref, b_ref):
    inv = 1.0 / jnp.maximum(deg_ref[...], 1.0)
    mean = (aa_ref[0] + ab_ref[0]) * inv
    h = (jnp.dot(x_ref[...], ws_ref[...], preferred_element_type=jnp.float32)
         + jnp.dot(mean, wn_ref[...], preferred_element_type=jnp.float32)
         + b_ref[...])
    return jnp.maximum(h, 0.0)


def _dense1_body(x_ref, aa_ref, ab_ref, deg_ref, ws_ref, wn_ref, b_ref,
                 o_ref):
    o_ref[...] = _mean_h(x_ref, aa_ref, ab_ref, deg_ref, ws_ref, wn_ref,
                         b_ref)


def _dense2_body(x_ref, aa_ref, ab_ref, deg_ref, ws_ref, wn_ref, b_ref,
                 wc_ref, bc_ref, o_ref):
    h = _mean_h(x_ref, aa_ref, ab_ref, deg_ref, ws_ref, wn_ref, b_ref)
    o_ref[...] = (jnp.dot(h, wc_ref[...], preferred_element_type=jnp.float32)
                  + bc_ref[...])


def _dense(body, n_rows, d, h, extra_w, t_out=False, r=2000):
    grid = n_rows // r
    assert n_rows % r == 0
    row_spec = pl.BlockSpec((r, d), lambda i: (i, 0))
    agg_a = pl.BlockSpec((1, r, d), lambda i: (0, i, 0))
    agg_b = pl.BlockSpec((1, r, d), lambda i: (1, i, 0))
    full = lambda shape: pl.BlockSpec(shape, lambda i: (0,) * len(shape))
    in_specs = [row_spec, agg_a, agg_b,
                pl.BlockSpec((r, 1), lambda i: (i, 0)),
                full((d, h)), full((d, h)), full((1, h))]
    out_d = h
    for w in extra_w:
        in_specs += [full(w[0]), full(w[1])]
        out_d = w[0][1]
    if t_out:
        out_specs = pl.BlockSpec((out_d, r), lambda i: (0, i))
        out_shape = jax.ShapeDtypeStruct((out_d, n_rows), jnp.float32)
    else:
        out_specs = pl.BlockSpec((r, out_d), lambda i: (i, 0))
        out_shape = jax.ShapeDtypeStruct((n_rows, out_d), jnp.float32)
    return pl.pallas_call(
        body,
        grid=(grid,),
        in_specs=in_specs,
        out_specs=out_specs,
        out_shape=out_shape,
    )


def kernel(x, edge_index, W1_self, W1_neigh, b1, W2_self, W2_neigh, b2,
           Wc, bc):
    n, d = x.shape
    h_dim = W1_self.shape[1]
    o_dim = W2_self.shape[1]
    c_dim = Wc.shape[1]
    e = edge_index.shape[1]

    n_pad = -(-n // 2048) * 2048
    assert e % _CH == 0
    edges = edge_index.astype(jnp.int32)

    agg1, degp = _make_sc_agg(n_pad, d, e, True)(x, edges)
    deg = degp.reshape(-1, n_pad).sum(axis=0).reshape(n_pad, 1)

    h1 = _dense(_dense1_body, n, d, h_dim, [])(
        x, agg1, agg1, deg, W1_self, W1_neigh,
        b1.reshape(1, h_dim))

    (agg2,) = _make_sc_agg(n_pad, h_dim, e, False)(h1, edges)

    out = _dense(_dense2_body, n, h_dim, o_dim,
                 [((o_dim, c_dim), (1, c_dim))])(
        h1, agg2, agg2, deg, W2_self, W2_neigh,
        b2.reshape(1, o_dim), Wc, bc.reshape(1, c_dim))
    return out
